# baseline (speedup 1.0000x reference)
import jax
import jax.numpy as jnp
from jax.experimental import pallas as pl

B = 4
E = 16
C = 1024
K = E // 2
NB = 8
BC = C // NB


def _gate(x):
    rs = jnp.sum(x, axis=-1)
    acc = jnp.zeros((B,), jnp.float32)
    cur = rs
    iota = jax.lax.broadcasted_iota(jnp.int32, (B, E), 1)
    for _ in range(K):
        m = jnp.max(cur, axis=1)
        acc = acc + m
        is_max = cur == m[:, None]
        first = jnp.min(jnp.where(is_max, iota, E), axis=1)
        cur = jnp.where(iota == first[:, None], -jnp.inf, cur)
    return acc * (1.0 / K)


def _moe_kernel(x_ref, wv_ref, wo_ref, out_ref):
    i = pl.program_id(0)
    x0 = x_ref[:, 0, :]
    v = jnp.dot(x0, wv_ref[...], preferred_element_type=jnp.float32)
    contrib = jnp.dot(v, wo_ref[...], preferred_element_type=jnp.float32)

    @pl.when(i == 0)
    def _():
        out_ref[...] = jnp.zeros_like(out_ref)

    @pl.when(i < NB - 1)
    def _():
        out_ref[...] += contrib

    @pl.when(i == NB - 1)
    def _():
        g = _gate(x_ref[...])
        out_ref[...] = (out_ref[...] + contrib) * g[:, None]


def kernel(x, q, Wq, Wk, Wv, Wo):
    out = pl.pallas_call(
        _moe_kernel,
        grid=(NB,),
        in_specs=[
            pl.BlockSpec((B, E, C), lambda i: (0, 0, 0)),
            pl.BlockSpec((C, BC), lambda i: (0, i)),
            pl.BlockSpec((BC, C), lambda i: (i, 0)),
        ],
        out_specs=pl.BlockSpec((B, C), lambda i: (0, 0)),
        out_shape=jax.ShapeDtypeStruct((B, C), jnp.float32),
    )(x, Wv[0], Wo[0])
    return out[:, None, :]


# speedup vs baseline: 1.7266x; 1.7266x over previous
import jax
import jax.numpy as jnp
from jax.experimental import pallas as pl

B = 4
E = 16
C = 1024
K = E // 2
NB = 8
BC = C // NB


def _gate(x):
    rs = jnp.sum(x, axis=-1)
    acc = jnp.zeros((B,), jnp.float32)
    cur = rs
    iota = jax.lax.broadcasted_iota(jnp.int32, (B, E), 1)
    for _ in range(K):
        m = jnp.max(cur, axis=1)
        acc = acc + m
        is_max = cur == m[:, None]
        first = jnp.min(jnp.where(is_max, iota, E), axis=1)
        cur = jnp.where(iota == first[:, None], -jnp.inf, cur)
    return acc * (1.0 / K)


def _moe_kernel(x_ref, wv_ref, wo_ref, out_ref):
    i = pl.program_id(0)
    x0 = x_ref[:, 0, :]
    v = jnp.dot(x0, wv_ref[0], preferred_element_type=jnp.float32)
    contrib = jnp.dot(v, wo_ref[0], preferred_element_type=jnp.float32)

    @pl.when(i == 0)
    def _():
        out_ref[...] = jnp.zeros_like(out_ref)

    @pl.when(i < NB - 1)
    def _():
        out_ref[...] += contrib

    @pl.when(i == NB - 1)
    def _():
        g = _gate(x_ref[...])
        out_ref[...] = (out_ref[...] + contrib) * g[:, None]


def kernel(x, q, Wq, Wk, Wv, Wo):
    out = pl.pallas_call(
        _moe_kernel,
        grid=(NB,),
        in_specs=[
            pl.BlockSpec((B, E, C), lambda i: (0, 0, 0)),
            pl.BlockSpec((1, C, BC), lambda i: (0, 0, i)),
            pl.BlockSpec((1, BC, C), lambda i: (0, i, 0)),
        ],
        out_specs=pl.BlockSpec((B, C), lambda i: (0, 0)),
        out_shape=jax.ShapeDtypeStruct((B, C), jnp.float32),
    )(x, Wv, Wo)
    return out[:, None, :]
